# baseline (speedup 1.0000x reference)
import jax
import jax.numpy as jnp
from jax.experimental import pallas as pl
from jax.experimental.pallas import tpu as pltpu

_BB = 64


def _mask_kernel(n_ref, out_ref):
    i = pl.program_id(0)
    rows = jax.lax.broadcasted_iota(jnp.int32, (1, 100, 100), 1)

    def body(j, carry):
        n = n_ref[i * _BB + j]
        out_ref[pl.ds(j, 1)] = (rows < n).astype(jnp.float32)
        return carry

    jax.lax.fori_loop(0, _BB, body, 0)


def kernel(tensor_span):
    b = tensor_span.shape[0]
    n = tensor_span[:, 0]
    nb = b // _BB
    grid_spec = pltpu.PrefetchScalarGridSpec(
        num_scalar_prefetch=1,
        grid=(nb,),
        in_specs=[],
        out_specs=pl.BlockSpec((_BB, 100, 100), lambda i, n_s: (i, 0, 0)),
    )
    return pl.pallas_call(
        _mask_kernel,
        grid_spec=grid_spec,
        out_shape=jax.ShapeDtypeStruct((b, 100, 100), jnp.float32),
    )(n)


# speedup vs baseline: 1.0337x; 1.0337x over previous
import jax
import jax.numpy as jnp
from jax.experimental import pallas as pl
from jax.experimental.pallas import tpu as pltpu

_BB = 64


def _mask_kernel(n_ref, out_ref):
    i = pl.program_id(0)
    rows = jax.lax.broadcasted_iota(jnp.int32, (100, 100), 0)
    for j in range(_BB):
        out_ref[j] = (rows < n_ref[i * _BB + j]).astype(jnp.float32)


def kernel(tensor_span):
    b = tensor_span.shape[0]
    n = tensor_span[:, 0]
    nb = b // _BB
    grid_spec = pltpu.PrefetchScalarGridSpec(
        num_scalar_prefetch=1,
        grid=(nb,),
        in_specs=[],
        out_specs=pl.BlockSpec((_BB, 100, 100), lambda i, n_s: (i, 0, 0)),
    )
    return pl.pallas_call(
        _mask_kernel,
        grid_spec=grid_spec,
        out_shape=jax.ShapeDtypeStruct((b, 100, 100), jnp.float32),
    )(n)
